# Initial kernel scaffold; baseline (speedup 1.0000x reference)
import jax
import jax.numpy as jnp
from jax.experimental import pallas as pl


def kernel(base, time_raw, static, gap, station_id, year_emb, month_emb, day_emb, hour_emb, time_W, time_b, static_W, static_b, gap_W1, gap_b1, gap_W2, gap_b2, station_table, alpha_time, alpha_static, alpha_gap, alpha_station, ln_gamma, ln_beta):
    raise NotImplementedError("write your pallas kernel here")


# speedup vs baseline: 11.3103x; 11.3103x over previous
import functools

import jax
import jax.numpy as jnp
from jax import lax
from jax.experimental import pallas as pl
from jax.experimental.pallas import tpu as pltpu
from jax.experimental.pallas import tpu_sc as plsc


def _station_gather(station_table, station_id):
    _, d = station_table.shape
    b = station_id.shape[0]
    info = plsc.get_sparse_core_info()
    nc, ns = info.num_cores, info.num_subcores
    nw = nc * ns
    b_per_w = b // nw
    mesh = plsc.VectorSubcoreMesh(core_axis_name="c", subcore_axis_name="s")

    @functools.partial(
        pl.kernel,
        mesh=mesh,
        out_type=jax.ShapeDtypeStruct((b, d), jnp.float32),
        scratch_types=[
            pltpu.VMEM((b_per_w,), jnp.int32),
            pltpu.VMEM((b_per_w, d), jnp.float32),
            pltpu.SemaphoreType.DMA,
        ],
    )
    def gather_kernel(table_hbm, idx_hbm, out_hbm, idx_v, rows_v, sem):
        wid = lax.axis_index("s") * nc + lax.axis_index("c")
        start = wid * b_per_w
        pltpu.sync_copy(idx_hbm.at[pl.ds(start, b_per_w)], idx_v)
        pltpu.async_copy(table_hbm.at[idx_v], rows_v, sem).wait()
        pltpu.sync_copy(rows_v, out_hbm.at[pl.ds(start, b_per_w)])

    return gather_kernel(station_table, station_id)


def _tc_body(base_ref, tr_ref, gap_ref, static_ref, st_ref, efull_ref,
             time_w_ref, time_b_ref, static_w_ref, static_b_ref,
             gap_w1_ref, gap_b1_ref, gap_w2_ref, gap_b2_ref,
             gamma_ref, beta_ref, alphas_ref, out_ref, *,
             k, s_len, offs, sizes, n_pad, off_b):
    r = k * s_len
    a_t = alphas_ref[0]
    a_s = alphas_ref[1]
    a_g = alphas_ref[2]
    a_st = alphas_ref[3]

    tr = tr_ref[...]
    lane = lax.broadcasted_iota(jnp.int32, (r, 128), 1)
    sel = None
    for j in range(4):
        idx = jnp.clip(tr[:, j:j + 1], 0, sizes[j] - 1) + offs[j]
        hit = lane == idx
        sel = hit if sel is None else (sel | hit)
    rowb = lax.broadcasted_iota(jnp.int32, (r, 1), 0) // s_len + off_b
    sel = (sel | (lane == rowb)).astype(jnp.float32)

    proj = jnp.dot(efull_ref[...], time_w_ref[...],
                   preferred_element_type=jnp.float32)
    s = jnp.dot(static_ref[...], static_w_ref[...],
                preferred_element_type=jnp.float32)
    c = a_s * (s + static_b_ref[...]) + a_st * st_ref[...]
    parts = [a_t * proj, c]
    if n_pad:
        parts.append(jnp.zeros((n_pad, 128), jnp.float32))
    tab = jnp.concatenate(parts, axis=0)
    add = jnp.dot(sel, tab, preferred_element_type=jnp.float32)

    g1 = jnp.dot(gap_ref[...], gap_w1_ref[...],
                 preferred_element_type=jnp.float32) + gap_b1_ref[...]
    g1 = jnp.maximum(g1, 0.0)
    g = jnp.dot(g1, gap_w2_ref[...],
                preferred_element_type=jnp.float32) + gap_b2_ref[...]

    z = base_ref[...] + add + a_t * time_b_ref[...] + a_g * g
    mean = jnp.mean(z, axis=1, keepdims=True)
    zc = z - mean
    var = jnp.mean(zc * zc, axis=1, keepdims=True)
    out_ref[...] = zc * lax.rsqrt(var + 1e-5) * gamma_ref[...] + beta_ref[...]


def kernel(base, time_raw, static, gap, station_id, year_emb, month_emb,
           day_emb, hour_emb, time_W, time_b, static_W, static_b, gap_W1,
           gap_b1, gap_W2, gap_b2, station_table, alpha_time, alpha_static,
           alpha_gap, alpha_station, ln_gamma, ln_beta):
    b, s_len, d = base.shape
    f = year_emb.shape[1]
    sizes = (year_emb.shape[0], month_emb.shape[0], day_emb.shape[0],
             hour_emb.shape[0])
    offs = (0, sizes[0], sizes[0] + sizes[1], sizes[0] + sizes[1] + sizes[2])
    n_cal = offs[3] + sizes[3]
    off_b = ((n_cal + 7) // 8) * 8
    k = 16
    assert off_b + k <= 128 and b % k == 0
    n_pad = 128 - off_b - k
    r = k * s_len

    st = _station_gather(station_table, station_id.astype(jnp.int32))

    efull = jnp.zeros((off_b, 4 * f), jnp.float32)
    for j, tbl in enumerate((year_emb, month_emb, day_emb, hour_emb)):
        efull = efull.at[offs[j]:offs[j] + sizes[j], j * f:(j + 1) * f].set(tbl)

    base2 = base.reshape(b * s_len, d)
    tr2 = time_raw.astype(jnp.int32).reshape(b * s_len, 4)
    gap2 = gap.reshape(b * s_len, gap.shape[-1])
    alphas = jnp.stack([alpha_time, alpha_static, alpha_gap,
                        alpha_station]).astype(jnp.float32)
    row = lambda v: v.reshape(1, d)

    body = functools.partial(_tc_body, k=k, s_len=s_len, offs=offs,
                             sizes=sizes, n_pad=n_pad, off_b=off_b)
    blk = lambda shape: pl.BlockSpec(shape, lambda i: (0,) * len(shape))
    out2 = pl.pallas_call(
        body,
        grid=(b // k,),
        in_specs=[
            pl.BlockSpec((r, d), lambda i: (i, 0)),
            pl.BlockSpec((r, 4), lambda i: (i, 0)),
            pl.BlockSpec((r, gap.shape[-1]), lambda i: (i, 0)),
            pl.BlockSpec((k, static.shape[-1]), lambda i: (i, 0)),
            pl.BlockSpec((k, d), lambda i: (i, 0)),
            blk(efull.shape),
            blk(time_W.shape),
            blk((1, d)),
            blk(static_W.shape),
            blk((1, d)),
            blk(gap_W1.shape),
            blk((1, d)),
            blk(gap_W2.shape),
            blk((1, d)),
            blk((1, d)),
            blk((1, d)),
            pl.BlockSpec(memory_space=pltpu.SMEM),
        ],
        out_specs=pl.BlockSpec((r, d), lambda i: (i, 0)),
        out_shape=jax.ShapeDtypeStruct((b * s_len, d), jnp.float32),
        compiler_params=pltpu.CompilerParams(
            dimension_semantics=("arbitrary",)),
    )(base2, tr2, gap2, static, st, efull, time_W, row(time_b), static_W,
      row(static_b), gap_W1, row(gap_b1), gap_W2, row(gap_b2), row(ln_gamma),
      row(ln_beta), alphas)
    return out2.reshape(b, s_len, d)


# speedup vs baseline: 11.7176x; 1.0360x over previous
import functools

import jax
import jax.numpy as jnp
from jax import lax
from jax.experimental import pallas as pl
from jax.experimental.pallas import tpu as pltpu
from jax.experimental.pallas import tpu_sc as plsc


def _station_gather(station_table, station_id):
    _, d = station_table.shape
    b = station_id.shape[0]
    info = plsc.get_sparse_core_info()
    nc, ns = info.num_cores, info.num_subcores
    nw = nc * ns
    b_per_w = b // nw
    mesh = plsc.VectorSubcoreMesh(core_axis_name="c", subcore_axis_name="s")

    @functools.partial(
        pl.kernel,
        mesh=mesh,
        out_type=jax.ShapeDtypeStruct((b, d), jnp.float32),
        scratch_types=[
            pltpu.VMEM((b_per_w,), jnp.int32),
            pltpu.VMEM((b_per_w, d), jnp.float32),
            pltpu.SemaphoreType.DMA,
        ],
    )
    def gather_kernel(table_hbm, idx_hbm, out_hbm, idx_v, rows_v, sem):
        wid = lax.axis_index("s") * nc + lax.axis_index("c")
        start = wid * b_per_w
        pltpu.sync_copy(idx_hbm.at[pl.ds(start, b_per_w)], idx_v)
        pltpu.async_copy(table_hbm.at[idx_v], rows_v, sem).wait()
        pltpu.sync_copy(rows_v, out_hbm.at[pl.ds(start, b_per_w)])

    return gather_kernel(station_table, station_id)


def _tc_body(base_ref, tr_ref, gap_ref, static_ref, st_ref, efull_ref,
             time_w_ref, time_b_ref, static_w_ref, static_b_ref,
             gap_w1_ref, gap_b1_ref, gap_w2_ref, gap_b2_ref,
             gamma_ref, beta_ref, alphas_ref, out_ref, *,
             k, s_len, offs, sizes, n_pad, off_b):
    r = k * s_len
    a_t = alphas_ref[0]
    a_s = alphas_ref[1]
    a_g = alphas_ref[2]
    a_st = alphas_ref[3]

    tr = tr_ref[...]
    lane = lax.broadcasted_iota(jnp.int32, (r, 128), 1)
    sel = None
    for j in range(4):
        idx = jnp.clip(tr[:, j:j + 1], 0, sizes[j] - 1) + offs[j]
        hit = lane == idx
        sel = hit if sel is None else (sel | hit)
    rowb = lax.broadcasted_iota(jnp.int32, (r, 1), 0) // s_len + off_b
    sel = sel | (lane == rowb)

    proj = jnp.dot(efull_ref[...], time_w_ref[...],
                   preferred_element_type=jnp.float32)
    s = jnp.dot(static_ref[...], static_w_ref[...],
                preferred_element_type=jnp.float32)
    c = (a_s * (s + static_b_ref[...]) + a_st * st_ref[...]
         + a_t * time_b_ref[...] + a_g * gap_b2_ref[...])
    parts = [a_t * proj, c]
    if n_pad:
        parts.append(jnp.zeros((n_pad, 128), jnp.float32))
    tab = jnp.concatenate(parts, axis=0).astype(jnp.bfloat16)
    add = jnp.dot(sel.astype(jnp.bfloat16), tab,
                  preferred_element_type=jnp.float32)

    g1 = jnp.dot(gap_ref[...], gap_w1_ref[...],
                 preferred_element_type=jnp.float32) + gap_b1_ref[...]
    g1 = jnp.maximum(g1, 0.0).astype(jnp.bfloat16)
    w2 = (a_g * gap_w2_ref[...]).astype(jnp.bfloat16)
    g = jnp.dot(g1, w2, preferred_element_type=jnp.float32)

    z = base_ref[...] + add + g
    mean = jnp.mean(z, axis=1, keepdims=True)
    zc = z - mean
    var = jnp.mean(zc * zc, axis=1, keepdims=True)
    out_ref[...] = zc * lax.rsqrt(var + 1e-5) * gamma_ref[...] + beta_ref[...]


def kernel(base, time_raw, static, gap, station_id, year_emb, month_emb,
           day_emb, hour_emb, time_W, time_b, static_W, static_b, gap_W1,
           gap_b1, gap_W2, gap_b2, station_table, alpha_time, alpha_static,
           alpha_gap, alpha_station, ln_gamma, ln_beta):
    b, s_len, d = base.shape
    f = year_emb.shape[1]
    sizes = (year_emb.shape[0], month_emb.shape[0], day_emb.shape[0],
             hour_emb.shape[0])
    offs = (0, sizes[0], sizes[0] + sizes[1], sizes[0] + sizes[1] + sizes[2])
    n_cal = offs[3] + sizes[3]
    off_b = ((n_cal + 7) // 8) * 8
    k = 32
    assert off_b + k <= 128 and b % k == 0
    n_pad = 128 - off_b - k
    r = k * s_len

    st = _station_gather(station_table, station_id.astype(jnp.int32))

    efull = jnp.zeros((off_b, 4 * f), jnp.float32)
    for j, tbl in enumerate((year_emb, month_emb, day_emb, hour_emb)):
        efull = efull.at[offs[j]:offs[j] + sizes[j], j * f:(j + 1) * f].set(tbl)

    base2 = base.reshape(b * s_len, d)
    tr2 = time_raw.astype(jnp.int32).reshape(b * s_len, 4)
    gap2 = gap.reshape(b * s_len, gap.shape[-1])
    alphas = jnp.stack([alpha_time, alpha_static, alpha_gap,
                        alpha_station]).astype(jnp.float32)
    row = lambda v: v.reshape(1, d)

    body = functools.partial(_tc_body, k=k, s_len=s_len, offs=offs,
                             sizes=sizes, n_pad=n_pad, off_b=off_b)
    blk = lambda shape: pl.BlockSpec(shape, lambda i: (0,) * len(shape))
    out2 = pl.pallas_call(
        body,
        grid=(b // k,),
        in_specs=[
            pl.BlockSpec((r, d), lambda i: (i, 0)),
            pl.BlockSpec((r, 4), lambda i: (i, 0)),
            pl.BlockSpec((r, gap.shape[-1]), lambda i: (i, 0)),
            pl.BlockSpec((k, static.shape[-1]), lambda i: (i, 0)),
            pl.BlockSpec((k, d), lambda i: (i, 0)),
            blk(efull.shape),
            blk(time_W.shape),
            blk((1, d)),
            blk(static_W.shape),
            blk((1, d)),
            blk(gap_W1.shape),
            blk((1, d)),
            blk(gap_W2.shape),
            blk((1, d)),
            blk((1, d)),
            blk((1, d)),
            pl.BlockSpec(memory_space=pltpu.SMEM),
        ],
        out_specs=pl.BlockSpec((r, d), lambda i: (i, 0)),
        out_shape=jax.ShapeDtypeStruct((b * s_len, d), jnp.float32),
        compiler_params=pltpu.CompilerParams(
            dimension_semantics=("arbitrary",)),
    )(base2, tr2, gap2, static, st, efull, time_W, row(time_b), static_W,
      row(static_b), gap_W1, row(gap_b1), gap_W2, row(gap_b2), row(ln_gamma),
      row(ln_beta), alphas)
    return out2.reshape(b, s_len, d)


# speedup vs baseline: 11.7235x; 1.0005x over previous
import functools

import jax
import jax.numpy as jnp
from jax import lax
from jax.experimental import pallas as pl
from jax.experimental.pallas import tpu as pltpu
from jax.experimental.pallas import tpu_sc as plsc


def _station_gather(station_table, station_id):
    _, d = station_table.shape
    b = station_id.shape[0]
    info = plsc.get_sparse_core_info()
    nc, ns = info.num_cores, info.num_subcores
    nw = nc * ns
    b_per_w = b // nw
    mesh = plsc.VectorSubcoreMesh(core_axis_name="c", subcore_axis_name="s")

    @functools.partial(
        pl.kernel,
        mesh=mesh,
        out_type=jax.ShapeDtypeStruct((b, d), jnp.float32),
        compiler_params=pltpu.CompilerParams(use_tc_tiling_on_sc=True),
        scratch_types=[
            pltpu.VMEM((b_per_w,), jnp.int32),
            pltpu.VMEM((b_per_w, d), jnp.float32),
            pltpu.SemaphoreType.DMA,
        ],
    )
    def gather_kernel(table_hbm, idx_hbm, out_hbm, idx_v, rows_v, sem):
        wid = lax.axis_index("s") * nc + lax.axis_index("c")
        start = wid * b_per_w
        pltpu.sync_copy(idx_hbm.at[pl.ds(start, b_per_w)], idx_v)
        pltpu.async_copy(table_hbm.at[idx_v], rows_v, sem).wait()
        pltpu.sync_copy(rows_v, out_hbm.at[pl.ds(start, b_per_w)])

    return gather_kernel(station_table, station_id)


def _tc_body(base_ref, tr_ref, gap_ref, static_ref, st_ref, efull_ref,
             time_w_ref, time_b_ref, static_w_ref, static_b_ref,
             gap_w1_ref, gap_b1_ref, gap_w2_ref, gap_b2_ref,
             gamma_ref, beta_ref, alphas_ref, out_ref, *,
             k, s_len, offs, sizes, n_pad, off_b):
    r = k * s_len
    a_t = alphas_ref[0]
    a_s = alphas_ref[1]
    a_g = alphas_ref[2]
    a_st = alphas_ref[3]

    tr = tr_ref[...]
    lane = lax.broadcasted_iota(jnp.int32, (r, 128), 1)
    sel = None
    for j in range(4):
        idx = jnp.clip(tr[:, j:j + 1], 0, sizes[j] - 1) + offs[j]
        hit = lane == idx
        sel = hit if sel is None else (sel | hit)
    rowb = lax.broadcasted_iota(jnp.int32, (r, 1), 0) // s_len + off_b
    sel = sel | (lane == rowb)

    proj = jnp.dot(efull_ref[...], time_w_ref[...],
                   preferred_element_type=jnp.float32)
    s = jnp.dot(static_ref[...], static_w_ref[...],
                preferred_element_type=jnp.float32)
    c = (a_s * (s + static_b_ref[...]) + a_st * st_ref[...]
         + a_t * time_b_ref[...] + a_g * gap_b2_ref[...])
    parts = [a_t * proj, c]
    if n_pad:
        parts.append(jnp.zeros((n_pad, 128), jnp.float32))
    tab = jnp.concatenate(parts, axis=0).astype(jnp.bfloat16)
    add = jnp.dot(sel.astype(jnp.bfloat16), tab,
                  preferred_element_type=jnp.float32)

    g1 = jnp.dot(gap_ref[...], gap_w1_ref[...],
                 preferred_element_type=jnp.float32) + gap_b1_ref[...]
    g1 = jnp.maximum(g1, 0.0).astype(jnp.bfloat16)
    w2 = (a_g * gap_w2_ref[...]).astype(jnp.bfloat16)
    g = jnp.dot(g1, w2, preferred_element_type=jnp.float32)

    z = base_ref[...] + add + g
    mean = jnp.mean(z, axis=1, keepdims=True)
    zc = z - mean
    var = jnp.mean(zc * zc, axis=1, keepdims=True)
    out_ref[...] = zc * lax.rsqrt(var + 1e-5) * gamma_ref[...] + beta_ref[...]


def kernel(base, time_raw, static, gap, station_id, year_emb, month_emb,
           day_emb, hour_emb, time_W, time_b, static_W, static_b, gap_W1,
           gap_b1, gap_W2, gap_b2, station_table, alpha_time, alpha_static,
           alpha_gap, alpha_station, ln_gamma, ln_beta):
    b, s_len, d = base.shape
    f = year_emb.shape[1]
    sizes = (year_emb.shape[0], month_emb.shape[0], day_emb.shape[0],
             hour_emb.shape[0])
    offs = (0, sizes[0], sizes[0] + sizes[1], sizes[0] + sizes[1] + sizes[2])
    n_cal = offs[3] + sizes[3]
    off_b = ((n_cal + 7) // 8) * 8
    k = 32
    assert off_b + k <= 128 and b % k == 0
    n_pad = 128 - off_b - k
    r = k * s_len

    st = _station_gather(station_table, station_id.astype(jnp.int32))

    efull = jnp.zeros((off_b, 4 * f), jnp.float32)
    for j, tbl in enumerate((year_emb, month_emb, day_emb, hour_emb)):
        efull = efull.at[offs[j]:offs[j] + sizes[j], j * f:(j + 1) * f].set(tbl)

    base2 = base.reshape(b * s_len, d)
    tr2 = time_raw.astype(jnp.int32).reshape(b * s_len, 4)
    gap2 = gap.reshape(b * s_len, gap.shape[-1])
    alphas = jnp.stack([alpha_time, alpha_static, alpha_gap,
                        alpha_station]).astype(jnp.float32)
    row = lambda v: v.reshape(1, d)

    body = functools.partial(_tc_body, k=k, s_len=s_len, offs=offs,
                             sizes=sizes, n_pad=n_pad, off_b=off_b)
    blk = lambda shape: pl.BlockSpec(shape, lambda i: (0,) * len(shape))
    out2 = pl.pallas_call(
        body,
        grid=(b // k,),
        in_specs=[
            pl.BlockSpec((r, d), lambda i: (i, 0)),
            pl.BlockSpec((r, 4), lambda i: (i, 0)),
            pl.BlockSpec((r, gap.shape[-1]), lambda i: (i, 0)),
            pl.BlockSpec((k, static.shape[-1]), lambda i: (i, 0)),
            pl.BlockSpec((k, d), lambda i: (i, 0)),
            blk(efull.shape),
            blk(time_W.shape),
            blk((1, d)),
            blk(static_W.shape),
            blk((1, d)),
            blk(gap_W1.shape),
            blk((1, d)),
            blk(gap_W2.shape),
            blk((1, d)),
            blk((1, d)),
            blk((1, d)),
            pl.BlockSpec(memory_space=pltpu.SMEM),
        ],
        out_specs=pl.BlockSpec((r, d), lambda i: (i, 0)),
        out_shape=jax.ShapeDtypeStruct((b * s_len, d), jnp.float32),
        compiler_params=pltpu.CompilerParams(
            dimension_semantics=("arbitrary",)),
    )(base2, tr2, gap2, static, st, efull, time_W, row(time_b), static_W,
      row(static_b), gap_W1, row(gap_b1), gap_W2, row(gap_b2), row(ln_gamma),
      row(ln_beta), alphas)
    return out2.reshape(b, s_len, d)
